# baseline (speedup 1.0000x reference)
import jax
import jax.numpy as jnp
from jax.experimental import pallas as pl
from jax.experimental.pallas import tpu as pltpu

MAX_SEQ = 8192
SEQ = 512
COLS = 8 * 128
BLK = 512


def _body(s_ref, kc_ref, vc_ref, kv_ref, vv_ref, ko_ref, vo_ref):
    i = pl.program_id(0)
    start = s_ref[0]
    b0 = i * BLK
    overlap = jnp.logical_and(b0 < start + SEQ, b0 + BLK > start)

    @pl.when(overlap)
    def _():
        rows = b0 + jax.lax.broadcasted_iota(jnp.int32, (BLK, 1), 0)
        mask = jnp.logical_and(rows >= start, rows < start + SEQ)
        shift = jax.lax.rem(start - b0, SEQ)
        kv = pltpu.roll(kv_ref[...], shift, 0)
        vv = pltpu.roll(vv_ref[...], shift, 0)
        ko_ref[...] = jnp.where(mask, kv, kc_ref[...])
        vo_ref[...] = jnp.where(mask, vv, vc_ref[...])

    @pl.when(jnp.logical_not(overlap))
    def _():
        ko_ref[...] = kc_ref[...]
        vo_ref[...] = vc_ref[...]


def kernel(input_pos, k_val, v_val, k_cache, v_cache):
    shp = k_cache.shape
    kc = k_cache.reshape(MAX_SEQ, COLS)
    vc = v_cache.reshape(MAX_SEQ, COLS)
    kv = k_val.reshape(SEQ, COLS)
    vv = v_val.reshape(SEQ, COLS)
    start = jnp.clip(input_pos[0], 0, MAX_SEQ - SEQ).reshape(1).astype(jnp.int32)

    grid_spec = pltpu.PrefetchScalarGridSpec(
        num_scalar_prefetch=1,
        grid=(MAX_SEQ // BLK,),
        in_specs=[
            pl.BlockSpec((BLK, COLS), lambda i, s: (i, 0)),
            pl.BlockSpec((BLK, COLS), lambda i, s: (i, 0)),
            pl.BlockSpec((SEQ, COLS), lambda i, s: (0, 0)),
            pl.BlockSpec((SEQ, COLS), lambda i, s: (0, 0)),
        ],
        out_specs=[
            pl.BlockSpec((BLK, COLS), lambda i, s: (i, 0)),
            pl.BlockSpec((BLK, COLS), lambda i, s: (i, 0)),
        ],
    )
    ko, vo = pl.pallas_call(
        _body,
        grid_spec=grid_spec,
        out_shape=[
            jax.ShapeDtypeStruct((MAX_SEQ, COLS), jnp.float32),
            jax.ShapeDtypeStruct((MAX_SEQ, COLS), jnp.float32),
        ],
        compiler_params=pltpu.CompilerParams(
            dimension_semantics=("arbitrary",),
        ),
    )(start, kc, vc, kv, vv)
    return (ko.reshape(shp), vo.reshape(shp))
